# baseline (speedup 1.0000x reference)
import jax
import jax.numpy as jnp
from jax.experimental import pallas as pl
from jax.experimental.pallas import tpu as pltpu

_BLK1 = 32
_BLK2 = 64


def _dot_t(a, b):
    return jax.lax.dot_general(a, b, (((1,), (1,)), ((), ())),
                               preferred_element_type=jnp.float32)


_CHUNK = 32


def _pass1_body(x_ref, uw_ref, ub_ref, vw_ref, vb_ref, h_ref, s_ref, s2_ref):
    B, n, c = x_ref.shape
    x = x_ref[...]
    xf = x.reshape(B * n, c)
    vx = (_dot_t(xf, vw_ref[...]) + vb_ref[...]).reshape(B, n, c)
    ux = (_dot_t(xf, uw_ref[...]) + ub_ref[...]).reshape(B, n, c)

    s_part = jnp.zeros((n, 1), jnp.float32)
    s2_part = jnp.zeros((n, 1), jnp.float32)
    neg = jnp.float32(-jnp.inf)
    nf = jnp.float32(n)
    for i in range(B):
        xs = x[i]
        si = _dot_t(xs, xs)
        m1 = jnp.max(si, axis=0, keepdims=True)
        lt1 = si < m1
        t = jnp.where(lt1, si, neg)
        m2 = jnp.max(t, axis=0, keepdims=True)
        lt2 = t < m2
        t = jnp.where(lt2, t, neg)
        m3 = jnp.max(t, axis=0, keepdims=True)
        lt3 = t < m3
        t = jnp.where(lt3, t, neg)
        m4 = jnp.max(t, axis=0, keepdims=True)
        p1 = jnp.sum(lt1.astype(jnp.float32), axis=0, keepdims=True)
        p2 = p1 + jnp.sum(lt2.astype(jnp.float32), axis=0, keepdims=True)
        p3 = p2 + jnp.sum(lt3.astype(jnp.float32), axis=0, keepdims=True)
        thr = jnp.where(p1 <= n - 4.0, m1,
                        jnp.where(p2 <= 2.0 * n - 4.0, m2,
                                  jnp.where(p3 <= 3.0 * n - 4.0, m3, m4)))
        ltf = si < thr
        deg = nf - jnp.sum(ltf.astype(jnp.float32), axis=0, keepdims=True)
        dinv = jax.lax.rsqrt(deg)
        a = jnp.where(ltf, jnp.float32(0.0), dinv)
        dinv_r = dinv.reshape(n, 1)
        agg = jax.lax.dot_general(a, dinv_r * vx[i],
                                  (((0,), (0,)), ((), ())),
                                  preferred_element_type=jnp.float32)
        h = agg + ux[i]
        h_ref[i] = h
        s_part = s_part + jnp.sum(h, axis=1, keepdims=True)
        s2_part = s2_part + jnp.sum(h * h, axis=1, keepdims=True)

    @pl.when(pl.program_id(0) == 0)
    def _init():
        s_ref[...] = jnp.zeros_like(s_ref)
        s2_ref[...] = jnp.zeros_like(s2_ref)

    s_ref[...] += s_part
    s2_ref[...] += s2_part


def _pass2_body(x_ref, h_ref, sc_ref, sh_ref, o_ref):
    _, n, _ = x_ref.shape
    sc = sc_ref[...].reshape(1, n, 1)
    sh = sh_ref[...].reshape(1, n, 1)
    o_ref[...] = jnp.maximum(x_ref[...] + h_ref[...] * sc + sh, 0.0)


def kernel(x, U_w, U_b, V_w, V_b, bn_w, bn_b):
    b, n, c = x.shape
    ub = U_b.reshape(1, c)
    vb = V_b.reshape(1, c)

    h, s, s2 = pl.pallas_call(
        _pass1_body,
        grid=(b // _BLK1,),
        in_specs=[
            pl.BlockSpec((_BLK1, n, c), lambda i: (i, 0, 0)),
            pl.BlockSpec((c, c), lambda i: (0, 0)),
            pl.BlockSpec((1, c), lambda i: (0, 0)),
            pl.BlockSpec((c, c), lambda i: (0, 0)),
            pl.BlockSpec((1, c), lambda i: (0, 0)),
        ],
        out_specs=[
            pl.BlockSpec((_BLK1, n, c), lambda i: (i, 0, 0)),
            pl.BlockSpec((n, 1), lambda i: (0, 0)),
            pl.BlockSpec((n, 1), lambda i: (0, 0)),
        ],
        out_shape=[
            jax.ShapeDtypeStruct((b, n, c), jnp.float32),
            jax.ShapeDtypeStruct((n, 1), jnp.float32),
            jax.ShapeDtypeStruct((n, 1), jnp.float32),
        ],
    )(x, U_w, ub, V_w, vb)

    denom = float(b * c)
    mean = s / denom
    var = s2 / denom - mean * mean
    scale = bn_w.reshape(n, 1) * jax.lax.rsqrt(var + 1e-5)
    shift = bn_b.reshape(n, 1) - mean * scale

    out = pl.pallas_call(
        _pass2_body,
        grid=(b // _BLK2,),
        in_specs=[
            pl.BlockSpec((_BLK2, n, c), lambda i: (i, 0, 0)),
            pl.BlockSpec((_BLK2, n, c), lambda i: (i, 0, 0)),
            pl.BlockSpec((n, 1), lambda i: (0, 0)),
            pl.BlockSpec((n, 1), lambda i: (0, 0)),
        ],
        out_specs=pl.BlockSpec((_BLK2, n, c), lambda i: (i, 0, 0)),
        out_shape=jax.ShapeDtypeStruct((b, n, c), jnp.float32),
    )(x, h, scale, shift)
    return out


# speedup vs baseline: 1.0603x; 1.0603x over previous
import jax
import jax.numpy as jnp
from jax.experimental import pallas as pl
from jax.experimental.pallas import tpu as pltpu

_BLK1 = 16
_BLK2 = 64


def _dot_t(a, b):
    return jax.lax.dot_general(a, b, (((1,), (1,)), ((), ())),
                               preferred_element_type=jnp.float32)


_CHUNK = 32


def _pass1_body(x_ref, uw_ref, ub_ref, vw_ref, vb_ref, h_ref, s_ref, s2_ref):
    B, n, c = x_ref.shape
    x = x_ref[...]
    xf = x.reshape(B * n, c)
    vx = (_dot_t(xf, vw_ref[...]) + vb_ref[...]).reshape(B, n, c)
    ux = (_dot_t(xf, uw_ref[...]) + ub_ref[...]).reshape(B, n, c)

    s_part = jnp.zeros((n, 1), jnp.float32)
    s2_part = jnp.zeros((n, 1), jnp.float32)
    neg = jnp.float32(-jnp.inf)
    nf = jnp.float32(n)
    for i in range(B):
        xs = x[i]
        si = _dot_t(xs, xs)
        m1 = jnp.max(si, axis=0, keepdims=True)
        lt1 = si < m1
        t = jnp.where(lt1, si, neg)
        m2 = jnp.max(t, axis=0, keepdims=True)
        lt2 = t < m2
        t = jnp.where(lt2, t, neg)
        m3 = jnp.max(t, axis=0, keepdims=True)
        lt3 = t < m3
        t = jnp.where(lt3, t, neg)
        m4 = jnp.max(t, axis=0, keepdims=True)
        p1 = jnp.sum(lt1.astype(jnp.float32), axis=0, keepdims=True)
        p2 = p1 + jnp.sum(lt2.astype(jnp.float32), axis=0, keepdims=True)
        p3 = p2 + jnp.sum(lt3.astype(jnp.float32), axis=0, keepdims=True)
        thr = jnp.where(p1 <= n - 4.0, m1,
                        jnp.where(p2 <= 2.0 * n - 4.0, m2,
                                  jnp.where(p3 <= 3.0 * n - 4.0, m3, m4)))
        ltf = si < thr
        deg = nf - jnp.sum(ltf.astype(jnp.float32), axis=0, keepdims=True)
        dinv = jax.lax.rsqrt(deg)
        a = jnp.where(ltf, jnp.float32(0.0), dinv)
        dinv_r = dinv.reshape(n, 1)
        agg = jax.lax.dot_general(a, dinv_r * vx[i],
                                  (((0,), (0,)), ((), ())),
                                  preferred_element_type=jnp.float32)
        h = agg + ux[i]
        h_ref[i] = h.astype(jnp.bfloat16)
        s_part = s_part + jnp.sum(h, axis=1, keepdims=True)
        s2_part = s2_part + jnp.sum(h * h, axis=1, keepdims=True)

    @pl.when(pl.program_id(0) == 0)
    def _init():
        s_ref[...] = jnp.zeros_like(s_ref)
        s2_ref[...] = jnp.zeros_like(s2_ref)

    s_ref[...] += s_part
    s2_ref[...] += s2_part


def _pass2_body(x_ref, h_ref, sc_ref, sh_ref, o_ref):
    _, n, _ = x_ref.shape
    sc = sc_ref[...].reshape(1, n, 1)
    sh = sh_ref[...].reshape(1, n, 1)
    o_ref[...] = jnp.maximum(
        x_ref[...] + h_ref[...].astype(jnp.float32) * sc + sh, 0.0)


def kernel(x, U_w, U_b, V_w, V_b, bn_w, bn_b):
    b, n, c = x.shape
    ub = U_b.reshape(1, c)
    vb = V_b.reshape(1, c)

    h, s, s2 = pl.pallas_call(
        _pass1_body,
        grid=(b // _BLK1,),
        in_specs=[
            pl.BlockSpec((_BLK1, n, c), lambda i: (i, 0, 0)),
            pl.BlockSpec((c, c), lambda i: (0, 0)),
            pl.BlockSpec((1, c), lambda i: (0, 0)),
            pl.BlockSpec((c, c), lambda i: (0, 0)),
            pl.BlockSpec((1, c), lambda i: (0, 0)),
        ],
        out_specs=[
            pl.BlockSpec((_BLK1, n, c), lambda i: (i, 0, 0)),
            pl.BlockSpec((n, 1), lambda i: (0, 0)),
            pl.BlockSpec((n, 1), lambda i: (0, 0)),
        ],
        out_shape=[
            jax.ShapeDtypeStruct((b, n, c), jnp.bfloat16),
            jax.ShapeDtypeStruct((n, 1), jnp.float32),
            jax.ShapeDtypeStruct((n, 1), jnp.float32),
        ],
    )(x, U_w, ub, V_w, vb)

    denom = float(b * c)
    mean = s / denom
    var = s2 / denom - mean * mean
    scale = bn_w.reshape(n, 1) * jax.lax.rsqrt(var + 1e-5)
    shift = bn_b.reshape(n, 1) - mean * scale

    out = pl.pallas_call(
        _pass2_body,
        grid=(b // _BLK2,),
        in_specs=[
            pl.BlockSpec((_BLK2, n, c), lambda i: (i, 0, 0)),
            pl.BlockSpec((_BLK2, n, c), lambda i: (i, 0, 0)),
            pl.BlockSpec((n, 1), lambda i: (0, 0)),
            pl.BlockSpec((n, 1), lambda i: (0, 0)),
        ],
        out_specs=pl.BlockSpec((_BLK2, n, c), lambda i: (i, 0, 0)),
        out_shape=jax.ShapeDtypeStruct((b, n, c), jnp.float32),
    )(x, h, scale, shift)
    return out


# speedup vs baseline: 1.0610x; 1.0007x over previous
import jax
import jax.numpy as jnp
from jax.experimental import pallas as pl

_BLK1 = 16
_BLK2 = 64


def _dot_t(a, b):
    return jax.lax.dot_general(a, b, (((1,), (1,)), ((), ())),
                               preferred_element_type=jnp.float32)


def _pass1_body(x_ref, uw_ref, ub_ref, vw_ref, vb_ref, h_ref, s_ref, s2_ref):
    B, n, c = x_ref.shape
    x = x_ref[...]
    xf = x.reshape(B * n, c)
    vx = (_dot_t(xf, vw_ref[...]) + vb_ref[...]).reshape(B, n, c)
    ux = (_dot_t(xf, uw_ref[...]) + ub_ref[...]).reshape(B, n, c)

    s_part = jnp.zeros((n, 1), jnp.float32)
    s2_part = jnp.zeros((n, 1), jnp.float32)
    neg = jnp.float32(-jnp.inf)
    nf = jnp.float32(n)
    for i in range(B):
        xs = x[i]
        si = _dot_t(xs, xs)
        m1 = jnp.max(si, axis=0, keepdims=True)
        lt1 = si < m1
        t = jnp.where(lt1, si, neg)
        m2 = jnp.max(t, axis=0, keepdims=True)
        lt2 = t < m2
        t = jnp.where(lt2, t, neg)
        m3 = jnp.max(t, axis=0, keepdims=True)
        lt3 = t < m3
        t = jnp.where(lt3, t, neg)
        m4 = jnp.max(t, axis=0, keepdims=True)
        p1 = jnp.sum(lt1.astype(jnp.float32), axis=0, keepdims=True)
        p2 = p1 + jnp.sum(lt2.astype(jnp.float32), axis=0, keepdims=True)
        p3 = p2 + jnp.sum(lt3.astype(jnp.float32), axis=0, keepdims=True)
        thr = jnp.where(p1 <= n - 4.0, m1,
                        jnp.where(p2 <= 2.0 * n - 4.0, m2,
                                  jnp.where(p3 <= 3.0 * n - 4.0, m3, m4)))
        ltf = si < thr
        deg = nf - jnp.sum(ltf.astype(jnp.float32), axis=0, keepdims=True)
        dinv = jax.lax.rsqrt(deg)
        a = jnp.where(ltf, jnp.float32(0.0), dinv)
        dinv_r = dinv.reshape(n, 1)
        agg = jax.lax.dot_general(a, dinv_r * vx[i],
                                  (((0,), (0,)), ((), ())),
                                  preferred_element_type=jnp.float32)
        h = agg + ux[i]
        h_ref[i] = h.astype(jnp.bfloat16)
        s_part = s_part + jnp.sum(h, axis=1, keepdims=True)
        s2_part = s2_part + jnp.sum(h * h, axis=1, keepdims=True)

    @pl.when(pl.program_id(0) == 0)
    def _init():
        s_ref[...] = jnp.zeros_like(s_ref)
        s2_ref[...] = jnp.zeros_like(s2_ref)

    s_ref[...] += s_part
    s2_ref[...] += s2_part


def _pass2_body(x_ref, h_ref, sc_ref, sh_ref, o_ref):
    _, n, _ = x_ref.shape
    sc = sc_ref[...].reshape(1, n, 1)
    sh = sh_ref[...].reshape(1, n, 1)
    o_ref[...] = jnp.maximum(
        x_ref[...] + h_ref[...].astype(jnp.float32) * sc + sh, 0.0)


def kernel(x, U_w, U_b, V_w, V_b, bn_w, bn_b):
    b, n, c = x.shape
    ub = U_b.reshape(1, c)
    vb = V_b.reshape(1, c)

    h, s, s2 = pl.pallas_call(
        _pass1_body,
        grid=(b // _BLK1,),
        in_specs=[
            pl.BlockSpec((_BLK1, n, c), lambda i: (i, 0, 0)),
            pl.BlockSpec((c, c), lambda i: (0, 0)),
            pl.BlockSpec((1, c), lambda i: (0, 0)),
            pl.BlockSpec((c, c), lambda i: (0, 0)),
            pl.BlockSpec((1, c), lambda i: (0, 0)),
        ],
        out_specs=[
            pl.BlockSpec((_BLK1, n, c), lambda i: (i, 0, 0)),
            pl.BlockSpec((n, 1), lambda i: (0, 0)),
            pl.BlockSpec((n, 1), lambda i: (0, 0)),
        ],
        out_shape=[
            jax.ShapeDtypeStruct((b, n, c), jnp.bfloat16),
            jax.ShapeDtypeStruct((n, 1), jnp.float32),
            jax.ShapeDtypeStruct((n, 1), jnp.float32),
        ],
    )(x, U_w, ub, V_w, vb)

    denom = float(b * c)
    mean = s / denom
    var = s2 / denom - mean * mean
    scale = bn_w.reshape(n, 1) * jax.lax.rsqrt(var + 1e-5)
    shift = bn_b.reshape(n, 1) - mean * scale

    out = pl.pallas_call(
        _pass2_body,
        grid=(b // _BLK2,),
        in_specs=[
            pl.BlockSpec((_BLK2, n, c), lambda i: (i, 0, 0)),
            pl.BlockSpec((_BLK2, n, c), lambda i: (i, 0, 0)),
            pl.BlockSpec((n, 1), lambda i: (0, 0)),
            pl.BlockSpec((n, 1), lambda i: (0, 0)),
        ],
        out_specs=pl.BlockSpec((_BLK2, n, c), lambda i: (i, 0, 0)),
        out_shape=jax.ShapeDtypeStruct((b, n, c), jnp.float32),
    )(x, h, scale, shift)
    return out
